# baseline (speedup 1.0000x reference)
import functools

import jax
import jax.numpy as jnp
from jax import lax
from jax.experimental import pallas as pl
from jax.experimental.pallas import tpu as pltpu
from jax.experimental.pallas import tpu_sc as plsc

D_MODEL = 128
MAX_LEN = 2048
HALF = MAX_LEN // 2
BATCH = 32
N_TABLES = 119
LANES = 16
R = 4
NG = D_MODEL // R
SLOTS = 4
UNROLL = 4

_MESH = plsc.VectorSubcoreMesh(core_axis_name="c", subcore_axis_name="s")


@functools.partial(
    pl.kernel,
    mesh=_MESH,
    out_type=jax.ShapeDtypeStruct((BATCH * D_MODEL, MAX_LEN), jnp.float32),
    scratch_types=(
        [
            pltpu.VMEM((NG, R), jnp.int32),
            pltpu.VMEM_SHARED((D_MODEL, HALF), jnp.float32),
        ]
        + [pltpu.VMEM((R, MAX_LEN), jnp.float32)] * SLOTS
        + [pltpu.VMEM((R, HALF), jnp.float32)] * (2 * SLOTS)
        + [pltpu.SemaphoreType.DMA] * (4 * SLOTS)
    ),
)
def _pe_add_sc(x_hbm, pea_hbm, peb_hbm, rowsb_hbm, out_hbm, idx_v,
               spa, *bufs_and_sems):
    xb = bufs_and_sems[0:SLOTS]
    pa = bufs_and_sems[SLOTS:2 * SLOTS]
    pb = bufs_and_sems[2 * SLOTS:3 * SLOTS]
    semx = bufs_and_sems[3 * SLOTS:4 * SLOTS]
    sema = bufs_and_sems[4 * SLOTS:5 * SLOTS]
    semb = bufs_and_sems[5 * SLOTS:6 * SLOTS]
    semo = bufs_and_sems[6 * SLOTS:7 * SLOTS]

    c = lax.axis_index("c")
    s = lax.axis_index("s")
    w = s * 2 + c

    @pl.when(s == 0)
    def _load_spmem():
        pltpu.sync_copy(pea_hbm, spa)

    pltpu.sync_copy(rowsb_hbm.at[w], idx_v)
    plsc.subcore_barrier()
    xrow0 = w * D_MODEL

    def prefetch(g, k):
        r0 = g * R
        pltpu.async_copy(x_hbm.at[pl.ds(xrow0 + r0, R)], xb[k], semx[k])
        pltpu.async_copy(spa.at[pl.ds(r0, R)], pa[k], sema[k])
        pltpu.async_copy(peb_hbm.at[idx_v.at[g]], pb[k], semb[k])

    def wait_in(k):
        pltpu.make_async_copy(x_hbm.at[pl.ds(0, R)], xb[k], semx[k]).wait()
        pltpu.make_async_copy(spa.at[pl.ds(0, R)], pa[k], sema[k]).wait()
        pltpu.make_async_copy(peb_hbm.at[pl.ds(0, R)], pb[k], semb[k]).wait()

    def drain_out(k):
        pltpu.make_async_copy(xb[k], out_hbm.at[pl.ds(0, R)], semo[k]).wait()

    def add(k):
        for r in range(R):
            def add_body(i, _, _r=r, _k=k):
                o = i * (LANES * UNROLL)
                for u in range(UNROLL):
                    o2 = o + u * LANES
                    plsc.addupdate(
                        xb[_k].at[_r, pl.ds(o2, LANES)],
                        pa[_k][_r, pl.ds(o2, LANES)],
                    )
                    plsc.addupdate(
                        xb[_k].at[_r, pl.ds(HALF + o2, LANES)],
                        pb[_k][_r, pl.ds(o2, LANES)],
                    )
                return 0
            lax.fori_loop(0, HALF // (LANES * UNROLL), add_body, 0)

    def consume(g, k):
        wait_in(k)
        add(k)
        pltpu.async_copy(xb[k], out_hbm.at[pl.ds(xrow0 + g * R, R)], semo[k])

    prefetch(0, 0)
    prefetch(1, 1)
    consume(0, 0)
    prefetch(2, 2)
    consume(1, 1)
    prefetch(3, 3)

    def body(i, _):
        g0 = i * SLOTS + 2
        for j, k in enumerate((2, 3, 0, 1)):
            g = g0 + j
            consume(g, k)
            k2 = (k + 2) % SLOTS
            drain_out(k2)
            prefetch(g + 2, k2)
        return 0

    lax.fori_loop(0, (NG - 4) // SLOTS, body, 0)

    consume(NG - 2, 2)
    drain_out(0)
    consume(NG - 1, 3)
    drain_out(1)
    drain_out(2)
    drain_out(3)


def kernel(x, pe, transition_len):
    tl = transition_len.astype(jnp.int32)
    r = jnp.arange(D_MODEL, dtype=jnp.int32)
    srcb = jnp.where(r[None, :] >= tl[:, None], r[None, :] - tl[:, None], r[None, :])
    pea = pe[0, :, :HALF]
    peb = pe[0, :, HALF:]
    out = _pe_add_sc(
        x.reshape(BATCH * D_MODEL, MAX_LEN),
        pea,
        peb,
        srcb.reshape(BATCH, NG, R),
    )
    return out.reshape(x.shape)


# speedup vs baseline: 1.2457x; 1.2457x over previous
import functools

import jax
import jax.numpy as jnp
from jax import lax
from jax.experimental import pallas as pl
from jax.experimental.pallas import tpu as pltpu
from jax.experimental.pallas import tpu_sc as plsc

D_MODEL = 128
MAX_LEN = 2048
HALF = MAX_LEN // 2
BATCH = 32
N_TABLES = 119
LANES = 16
R = 4
NG = D_MODEL // R
SLOTS = 4
UNROLL = 8

_MESH = plsc.VectorSubcoreMesh(core_axis_name="c", subcore_axis_name="s")


@functools.partial(
    pl.kernel,
    mesh=_MESH,
    out_type=jax.ShapeDtypeStruct((BATCH * D_MODEL, MAX_LEN), jnp.float32),
    scratch_types=(
        [
            pltpu.VMEM((NG, R), jnp.int32),
            pltpu.VMEM_SHARED((D_MODEL, HALF), jnp.float32),
        ]
        + [pltpu.VMEM((R, MAX_LEN), jnp.float32)] * (2 * SLOTS)
        + [pltpu.SemaphoreType.DMA] * (4 * SLOTS)
    ),
)
def _pe_add_sc(x_hbm, pea_hbm, peb_hbm, rowsb_hbm, out_hbm, idx_v,
               spa, *bufs_and_sems):
    xb = bufs_and_sems[0:SLOTS]
    pb = bufs_and_sems[SLOTS:2 * SLOTS]
    semx = bufs_and_sems[2 * SLOTS:3 * SLOTS]
    sema = bufs_and_sems[3 * SLOTS:4 * SLOTS]
    semb = bufs_and_sems[4 * SLOTS:5 * SLOTS]
    semo = bufs_and_sems[5 * SLOTS:6 * SLOTS]

    c = lax.axis_index("c")
    s = lax.axis_index("s")
    w = s * 2 + c

    @pl.when(s == 0)
    def _load_spmem():
        pltpu.sync_copy(pea_hbm, spa)

    pltpu.sync_copy(rowsb_hbm.at[w], idx_v)
    plsc.subcore_barrier()
    xrow0 = w * D_MODEL

    def prefetch(g, k):
        r0 = g * R
        pltpu.async_copy(x_hbm.at[pl.ds(xrow0 + r0, R)], xb[k], semx[k])
        pltpu.async_copy(
            spa.at[pl.ds(r0, R)],
            pb[k].at[pl.ds(0, R), pl.ds(0, HALF)],
            sema[k],
        )
        pltpu.async_copy(
            peb_hbm.at[idx_v.at[g]],
            pb[k].at[pl.ds(0, R), pl.ds(HALF, HALF)],
            semb[k],
        )

    def wait_in(k):
        pltpu.make_async_copy(x_hbm.at[pl.ds(0, R)], xb[k], semx[k]).wait()
        pltpu.make_async_copy(
            spa.at[pl.ds(0, R)],
            pb[k].at[pl.ds(0, R), pl.ds(0, HALF)],
            sema[k],
        ).wait()
        pltpu.make_async_copy(
            peb_hbm.at[pl.ds(0, R)],
            pb[k].at[pl.ds(0, R), pl.ds(HALF, HALF)],
            semb[k],
        ).wait()

    def drain_out(k):
        pltpu.make_async_copy(xb[k], out_hbm.at[pl.ds(0, R)], semo[k]).wait()

    def add(k):
        for r in range(R):
            def add_body(i, _, _r=r, _k=k):
                o = i * (LANES * UNROLL)
                for u in range(UNROLL):
                    o2 = o + u * LANES
                    plsc.addupdate(
                        xb[_k].at[_r, pl.ds(o2, LANES)],
                        pb[_k][_r, pl.ds(o2, LANES)],
                    )
                return 0
            lax.fori_loop(0, MAX_LEN // (LANES * UNROLL), add_body, 0)

    def consume(g, k):
        wait_in(k)
        add(k)
        pltpu.async_copy(xb[k], out_hbm.at[pl.ds(xrow0 + g * R, R)], semo[k])

    prefetch(0, 0)
    prefetch(1, 1)
    consume(0, 0)
    prefetch(2, 2)
    consume(1, 1)
    prefetch(3, 3)

    def body(i, _):
        g0 = i * SLOTS + 2
        for j, k in enumerate((2, 3, 0, 1)):
            g = g0 + j
            consume(g, k)
            k2 = (k + 2) % SLOTS
            drain_out(k2)
            prefetch(g + 2, k2)
        return 0

    lax.fori_loop(0, (NG - 4) // SLOTS, body, 0)

    consume(NG - 2, 2)
    drain_out(0)
    consume(NG - 1, 3)
    drain_out(1)
    drain_out(2)
    drain_out(3)


def kernel(x, pe, transition_len):
    tl = transition_len.astype(jnp.int32)
    r = jnp.arange(D_MODEL, dtype=jnp.int32)
    srcb = jnp.where(r[None, :] >= tl[:, None], r[None, :] - tl[:, None], r[None, :])
    pea = pe[0, :, :HALF]
    peb = pe[0, :, HALF:]
    out = _pe_add_sc(
        x.reshape(BATCH * D_MODEL, MAX_LEN),
        pea,
        peb,
        srcb.reshape(BATCH, NG, R),
    )
    return out.reshape(x.shape)


# speedup vs baseline: 1.3374x; 1.0736x over previous
import functools

import jax
import jax.numpy as jnp
from jax import lax
from jax.experimental import pallas as pl
from jax.experimental.pallas import tpu as pltpu
from jax.experimental.pallas import tpu_sc as plsc

D_MODEL = 128
MAX_LEN = 2048
BATCH = 32
N_TABLES = 119
LANES = 16
R = 4
NG = D_MODEL // R
SLOTS = 4
UNROLL = 8

_MESH = plsc.VectorSubcoreMesh(core_axis_name="c", subcore_axis_name="s")


@functools.partial(
    pl.kernel,
    mesh=_MESH,
    out_type=jax.ShapeDtypeStruct((BATCH * D_MODEL, MAX_LEN), jnp.float32),
    scratch_types=(
        [pltpu.VMEM((NG, R), jnp.int32)]
        + [pltpu.VMEM((R, MAX_LEN), jnp.float32)] * (2 * SLOTS)
        + [pltpu.SemaphoreType.DMA] * (3 * SLOTS)
    ),
)
def _pe_add_sc(x_hbm, pe_hbm, rows_hbm, out_hbm, idx_v, *bufs_and_sems):
    xb = bufs_and_sems[0:SLOTS]
    pb = bufs_and_sems[SLOTS:2 * SLOTS]
    semx = bufs_and_sems[2 * SLOTS:3 * SLOTS]
    semp = bufs_and_sems[3 * SLOTS:4 * SLOTS]
    semo = bufs_and_sems[4 * SLOTS:5 * SLOTS]

    c = lax.axis_index("c")
    s = lax.axis_index("s")
    w = s * 2 + c

    pltpu.sync_copy(rows_hbm.at[w], idx_v)
    xrow0 = w * D_MODEL

    def prefetch(g, k):
        pltpu.async_copy(pe_hbm.at[idx_v.at[g]], pb[k], semp[k])
        pltpu.async_copy(x_hbm.at[pl.ds(xrow0 + g * R, R)], xb[k], semx[k])

    def wait_in(k):
        pltpu.make_async_copy(x_hbm.at[pl.ds(0, R)], xb[k], semx[k]).wait()
        pltpu.make_async_copy(pe_hbm.at[pl.ds(0, R)], pb[k], semp[k]).wait()

    def drain_out(k):
        pltpu.make_async_copy(xb[k], out_hbm.at[pl.ds(0, R)], semo[k]).wait()

    def add(k):
        for r in range(R):
            def add_body(i, _, _r=r, _k=k):
                o = i * (LANES * UNROLL)
                for u in range(UNROLL):
                    o2 = o + u * LANES
                    plsc.addupdate(
                        xb[_k].at[_r, pl.ds(o2, LANES)],
                        pb[_k][_r, pl.ds(o2, LANES)],
                    )
                return 0
            lax.fori_loop(0, MAX_LEN // (LANES * UNROLL), add_body, 0)

    def consume(g, k):
        wait_in(k)
        add(k)
        pltpu.async_copy(xb[k], out_hbm.at[pl.ds(xrow0 + g * R, R)], semo[k])

    prefetch(0, 0)
    prefetch(1, 1)
    consume(0, 0)
    prefetch(2, 2)
    consume(1, 1)
    prefetch(3, 3)

    def body(i, _):
        g0 = i * SLOTS + 2
        for j, k in enumerate((2, 3, 0, 1)):
            g = g0 + j
            consume(g, k)
            k2 = (k + 2) % SLOTS
            drain_out(k2)
            prefetch(g + 2, k2)
        return 0

    lax.fori_loop(0, (NG - 4) // SLOTS, body, 0)

    consume(NG - 2, 2)
    drain_out(0)
    consume(NG - 1, 3)
    drain_out(1)
    drain_out(2)
    drain_out(3)


def kernel(x, pe, transition_len):
    tl = transition_len.astype(jnp.int32)
    rows = tl[:, None] * D_MODEL + jnp.arange(D_MODEL, dtype=jnp.int32)
    out = _pe_add_sc(
        x.reshape(BATCH * D_MODEL, MAX_LEN),
        pe.reshape(N_TABLES * D_MODEL, MAX_LEN),
        rows.reshape(BATCH, NG, R),
    )
    return out.reshape(x.shape)
